# baseline (device time: 19329 ns/iter reference)
import jax
import jax.numpy as jnp
from jax import lax
from jax.experimental import pallas as pl
from jax.experimental.pallas import tpu as pltpu


def kernel(partial, resid, gamma):
    _, m, d = partial.shape
    gamma2d = gamma.reshape(1, d)

    def body(p_ref, r_ref, g_ref, o_ref, comm_ref, send_sem, recv_sem):
        my_x = lax.axis_index("x")
        my_y = lax.axis_index("y")
        nbr = (1 - my_x, my_y)

        barrier_sem = pltpu.get_barrier_semaphore()
        pl.semaphore_signal(
            barrier_sem, inc=1, device_id=nbr,
            device_id_type=pl.DeviceIdType.MESH,
        )
        pl.semaphore_wait(barrier_sem, 1)

        rdma = pltpu.make_async_remote_copy(
            src_ref=p_ref.at[0],
            dst_ref=comm_ref,
            send_sem=send_sem,
            recv_sem=recv_sem,
            device_id=nbr,
            device_id_type=pl.DeviceIdType.MESH,
        )
        rdma.start()
        rdma.wait()

        y = p_ref[0] + comm_ref[...] + r_ref[...]
        rms = jnp.sqrt(jnp.mean(y * y, axis=-1, keepdims=True) + 1e-6)
        o_ref[...] = y / rms * g_ref[...]

    return pl.pallas_call(
        body,
        out_shape=jax.ShapeDtypeStruct((m, d), jnp.float32),
        in_specs=[
            pl.BlockSpec(memory_space=pltpu.VMEM),
            pl.BlockSpec(memory_space=pltpu.VMEM),
            pl.BlockSpec(memory_space=pltpu.VMEM),
        ],
        out_specs=pl.BlockSpec(memory_space=pltpu.VMEM),
        scratch_shapes=[
            pltpu.VMEM((m, d), jnp.float32),
            pltpu.SemaphoreType.DMA,
            pltpu.SemaphoreType.DMA,
        ],
        compiler_params=pltpu.CompilerParams(collective_id=0),
    )(partial, resid, gamma2d)


# device time: 17509 ns/iter; 1.1039x vs baseline; 1.1039x over previous
import jax
import jax.numpy as jnp
from jax import lax
from jax.experimental import pallas as pl
from jax.experimental.pallas import tpu as pltpu

C = 4


def kernel(partial, resid, gamma):
    _, m, d = partial.shape
    gamma2d = gamma.reshape(1, d)
    half = m // 2
    ch = half // C

    def body(p_ref, r_ref, g_ref, o_ref, comm_ref,
             send_a, recv_a, send_b, recv_b):
        my_x = lax.axis_index("x")
        my_y = lax.axis_index("y")
        xnbr = (1 - my_x, my_y)
        ynbr = (my_x, 1 - my_y)
        row0 = my_y * half

        barrier_sem = pltpu.get_barrier_semaphore()
        for nbr in (xnbr, ynbr):
            pl.semaphore_signal(
                barrier_sem, inc=1, device_id=nbr,
                device_id_type=pl.DeviceIdType.MESH,
            )
        pl.semaphore_wait(barrier_sem, 2)

        a_rdmas = []
        for c in range(C):
            a = pltpu.make_async_remote_copy(
                src_ref=p_ref.at[0, pl.ds(row0 + c * ch, ch), :],
                dst_ref=comm_ref.at[pl.ds(c * ch, ch), :],
                send_sem=send_a.at[c],
                recv_sem=recv_a.at[c],
                device_id=xnbr,
                device_id_type=pl.DeviceIdType.MESH,
            )
            a.start()
            a_rdmas.append(a)

        b_rdmas = []
        for c in range(C):
            a_rdmas[c].wait_recv()
            rows = pl.ds(row0 + c * ch, ch)
            y = p_ref[0, rows, :] + comm_ref[pl.ds(c * ch, ch), :] + r_ref[rows, :]
            rms = jnp.sqrt(jnp.mean(y * y, axis=-1, keepdims=True) + 1e-6)
            o_ref[rows, :] = y / rms * g_ref[...]
            b = pltpu.make_async_remote_copy(
                src_ref=o_ref.at[rows, :],
                dst_ref=o_ref.at[rows, :],
                send_sem=send_b.at[c],
                recv_sem=recv_b.at[c],
                device_id=ynbr,
                device_id_type=pl.DeviceIdType.MESH,
            )
            b.start()
            b_rdmas.append(b)

        for c in range(C):
            a_rdmas[c].wait_send()
            b_rdmas[c].wait_send()
            b_rdmas[c].wait_recv()

    return pl.pallas_call(
        body,
        out_shape=jax.ShapeDtypeStruct((m, d), jnp.float32),
        in_specs=[
            pl.BlockSpec(memory_space=pltpu.VMEM),
            pl.BlockSpec(memory_space=pltpu.VMEM),
            pl.BlockSpec(memory_space=pltpu.VMEM),
        ],
        out_specs=pl.BlockSpec(memory_space=pltpu.VMEM),
        scratch_shapes=[
            pltpu.VMEM((half, d), jnp.float32),
            pltpu.SemaphoreType.DMA((C,)),
            pltpu.SemaphoreType.DMA((C,)),
            pltpu.SemaphoreType.DMA((C,)),
            pltpu.SemaphoreType.DMA((C,)),
        ],
        compiler_params=pltpu.CompilerParams(collective_id=0),
    )(partial, resid, gamma2d)


# device time: 16972 ns/iter; 1.1389x vs baseline; 1.0316x over previous
import jax
import jax.numpy as jnp
from jax import lax
from jax.experimental import pallas as pl
from jax.experimental.pallas import tpu as pltpu

C = 8


def kernel(partial, resid, gamma):
    _, m, d = partial.shape
    gamma2d = gamma.reshape(1, d)
    half = m // 2
    ch = half // C

    def body(p_hbm, r_hbm, g_ref, o_ref, p_half, r_half, comm_ref,
             copy_sems, send_a, recv_a, send_b, recv_b):
        my_x = lax.axis_index("x")
        my_y = lax.axis_index("y")
        xnbr = (1 - my_x, my_y)
        ynbr = (my_x, 1 - my_y)
        row0 = my_y * half

        cp_p = pltpu.make_async_copy(
            p_hbm.at[0, pl.ds(row0, half), :], p_half, copy_sems.at[0])
        cp_r = pltpu.make_async_copy(
            r_hbm.at[pl.ds(row0, half), :], r_half, copy_sems.at[1])
        cp_p.start()
        cp_r.start()

        barrier_sem = pltpu.get_barrier_semaphore()
        for nbr in (xnbr, ynbr):
            pl.semaphore_signal(
                barrier_sem, inc=1, device_id=nbr,
                device_id_type=pl.DeviceIdType.MESH,
            )
        pl.semaphore_wait(barrier_sem, 2)
        cp_p.wait()

        a_rdmas = []
        for c in range(C):
            a = pltpu.make_async_remote_copy(
                src_ref=p_half.at[pl.ds(c * ch, ch), :],
                dst_ref=comm_ref.at[pl.ds(c * ch, ch), :],
                send_sem=send_a.at[c],
                recv_sem=recv_a.at[c],
                device_id=xnbr,
                device_id_type=pl.DeviceIdType.MESH,
            )
            a.start()
            a_rdmas.append(a)
        cp_r.wait()

        b_rdmas = []
        for c in range(C):
            a_rdmas[c].wait_recv()
            sl = pl.ds(c * ch, ch)
            y = p_half[sl, :] + comm_ref[sl, :] + r_half[sl, :]
            rs = lax.rsqrt(jnp.mean(y * y, axis=-1, keepdims=True) + 1e-6)
            rows = pl.ds(row0 + c * ch, ch)
            o_ref[rows, :] = y * rs * g_ref[...]
            b = pltpu.make_async_remote_copy(
                src_ref=o_ref.at[rows, :],
                dst_ref=o_ref.at[rows, :],
                send_sem=send_b.at[c],
                recv_sem=recv_b.at[c],
                device_id=ynbr,
                device_id_type=pl.DeviceIdType.MESH,
            )
            b.start()
            b_rdmas.append(b)

        for c in range(C):
            a_rdmas[c].wait_send()
            b_rdmas[c].wait_send()
            b_rdmas[c].wait_recv()

    return pl.pallas_call(
        body,
        out_shape=jax.ShapeDtypeStruct((m, d), jnp.float32),
        in_specs=[
            pl.BlockSpec(memory_space=pl.ANY),
            pl.BlockSpec(memory_space=pl.ANY),
            pl.BlockSpec(memory_space=pltpu.VMEM),
        ],
        out_specs=pl.BlockSpec(memory_space=pltpu.VMEM),
        scratch_shapes=[
            pltpu.VMEM((half, d), jnp.float32),
            pltpu.VMEM((half, d), jnp.float32),
            pltpu.VMEM((half, d), jnp.float32),
            pltpu.SemaphoreType.DMA((2,)),
            pltpu.SemaphoreType.DMA((C,)),
            pltpu.SemaphoreType.DMA((C,)),
            pltpu.SemaphoreType.DMA((C,)),
            pltpu.SemaphoreType.DMA((C,)),
        ],
        compiler_params=pltpu.CompilerParams(collective_id=0),
    )(partial, resid, gamma2d)


# device time: 16813 ns/iter; 1.1496x vs baseline; 1.0095x over previous
import jax
import jax.numpy as jnp
from jax import lax
from jax.experimental import pallas as pl
from jax.experimental.pallas import tpu as pltpu

_SIZES_B = (32, 32, 32, 32, 32, 32, 16)
_SIZES_L = (48, 48)
_SIZES = _SIZES_B + _SIZES_L
_NB = len(_SIZES_B)
_NC = len(_SIZES)


def kernel(partial, resid, gamma):
    _, m, d = partial.shape
    gamma2d = gamma.reshape(1, d)
    half = m // 2
    ov = _SIZES_L[1]
    nb_rows = sum(_SIZES_B)
    assert nb_rows + _SIZES_L[0] == half and _SIZES_L[0] == ov
    sel = half + ov
    offs = [sum(_SIZES[:c]) for c in range(_NC)]

    def body(p_hbm, r_hbm, g_ref, o_ref, p_sel, r_sel, comm_ref, out_v,
             copy_sems, store_sems, send_a, recv_a, send_b, recv_b):
        my_x = lax.axis_index("x")
        my_y = lax.axis_index("y")
        xnbr = (1 - my_x, my_y)
        ynbr = (my_x, 1 - my_y)
        row0 = pl.multiple_of(my_y * half, half)
        orow0 = pl.multiple_of((1 - my_y) * half, half)

        cps = [
            pltpu.make_async_copy(
                p_hbm.at[0, pl.ds(row0, half), :],
                p_sel.at[pl.ds(0, half), :], copy_sems.at[0]),
            pltpu.make_async_copy(
                p_hbm.at[0, pl.ds(orow0 + nb_rows, ov), :],
                p_sel.at[pl.ds(half, ov), :], copy_sems.at[1]),
            pltpu.make_async_copy(
                r_hbm.at[pl.ds(row0, half), :],
                r_sel.at[pl.ds(0, half), :], copy_sems.at[2]),
            pltpu.make_async_copy(
                r_hbm.at[pl.ds(orow0 + nb_rows, ov), :],
                r_sel.at[pl.ds(half, ov), :], copy_sems.at[3]),
        ]
        for cp in cps:
            cp.start()

        barrier_sem = pltpu.get_barrier_semaphore()
        for nbr in (xnbr, ynbr):
            pl.semaphore_signal(
                barrier_sem, inc=1, device_id=nbr,
                device_id_type=pl.DeviceIdType.MESH,
            )
        pl.semaphore_wait(barrier_sem, 2)
        cps[0].wait()
        cps[1].wait()

        a_rdmas = []
        for c in range(_NC):
            a = pltpu.make_async_remote_copy(
                src_ref=p_sel.at[pl.ds(offs[c], _SIZES[c]), :],
                dst_ref=comm_ref.at[pl.ds(offs[c], _SIZES[c]), :],
                send_sem=send_a.at[c],
                recv_sem=recv_a.at[c],
                device_id=xnbr,
                device_id_type=pl.DeviceIdType.MESH,
            )
            a.start()
            a_rdmas.append(a)
        cps[2].wait()
        cps[3].wait()

        ones = jnp.ones((d, 1), jnp.float32)
        b_rdmas = []
        stores = []
        for c in range(_NC):
            a_rdmas[c].wait_recv()
            sl = pl.ds(offs[c], _SIZES[c])
            y = p_sel[sl, :] + comm_ref[sl, :] + r_sel[sl, :]
            ssq = jax.lax.dot_general(
                y * y, ones, (((1,), (0,)), ((), ())),
                preferred_element_type=jnp.float32)
            rs = lax.rsqrt(ssq * (1.0 / d) + 1e-6)
            if c < _NC - 1:
                rows = pl.ds(row0 + offs[c], _SIZES[c])
            else:
                rows = pl.ds(orow0 + nb_rows, _SIZES[c])
            out_v[sl, :] = y * rs * g_ref[...]
            st = pltpu.make_async_copy(
                out_v.at[sl, :], o_ref.at[rows, :], store_sems.at[c])
            st.start()
            stores.append(st)
            if c < _NB:
                b = pltpu.make_async_remote_copy(
                    src_ref=out_v.at[sl, :],
                    dst_ref=o_ref.at[rows, :],
                    send_sem=send_b.at[c],
                    recv_sem=recv_b.at[c],
                    device_id=ynbr,
                    device_id_type=pl.DeviceIdType.MESH,
                )
                b.start()
                b_rdmas.append(b)

        for c in range(_NC):
            a_rdmas[c].wait_send()
            stores[c].wait()
        for c in range(_NB):
            b_rdmas[c].wait_send()
            b_rdmas[c].wait_recv()

    return pl.pallas_call(
        body,
        out_shape=jax.ShapeDtypeStruct((m, d), jnp.float32),
        in_specs=[
            pl.BlockSpec(memory_space=pl.ANY),
            pl.BlockSpec(memory_space=pl.ANY),
            pl.BlockSpec(memory_space=pltpu.VMEM),
        ],
        out_specs=pl.BlockSpec(memory_space=pl.ANY),
        scratch_shapes=[
            pltpu.VMEM((sel, d), jnp.float32),
            pltpu.VMEM((sel, d), jnp.float32),
            pltpu.VMEM((sel, d), jnp.float32),
            pltpu.VMEM((sel, d), jnp.float32),
            pltpu.SemaphoreType.DMA((4,)),
            pltpu.SemaphoreType.DMA((_NC,)),
            pltpu.SemaphoreType.DMA((_NC,)),
            pltpu.SemaphoreType.DMA((_NC,)),
            pltpu.SemaphoreType.DMA((_NB,)),
            pltpu.SemaphoreType.DMA((_NB,)),
        ],
        compiler_params=pltpu.CompilerParams(collective_id=0),
    )(partial, resid, gamma2d)
